# baseline (speedup 1.0000x reference)
import functools

import jax
import jax.numpy as jnp
from jax import lax
from jax.experimental import pallas as pl
from jax.experimental.pallas import tpu as pltpu
from jax.experimental.pallas import tpu_sc as plsc

B, P, L = 1024, 16, 3
E, H = 64, 64
NPATH = B * P
NCHUNK = 2
CPATH = NPATH // NCHUNK
CIDX = CPATH * L
CPAIR = CIDX // 2


_NC, _NS = 2, 16
NW = _NC * _NS
PER_W = CIDX // NW
CH = 128
NCH = PER_W // CH


def _sc_gather(table, idx1d):
    mesh = plsc.VectorSubcoreMesh(core_axis_name="c", subcore_axis_name="s")

    @functools.partial(
        pl.kernel,
        mesh=mesh,
        out_type=jax.ShapeDtypeStruct((CIDX, E), jnp.float32),
        scratch_types=[
            pltpu.VMEM((PER_W,), jnp.int32),
            pltpu.VMEM((PER_W, E), jnp.float32),
            pltpu.SemaphoreType.DMA,
        ],
        compiler_params=pltpu.CompilerParams(use_tc_tiling_on_sc=False),
    )
    def gather_k(table_hbm, idx_hbm, out_hbm, idx_v, rows_v, sem):
        wid = lax.axis_index("s") * _NC + lax.axis_index("c")
        pltpu.sync_copy(idx_hbm.at[pl.ds(wid * PER_W, PER_W)], idx_v)
        copies = []
        for j in range(NCH):
            copies.append(
                pltpu.async_copy(
                    table_hbm.at[idx_v.at[pl.ds(j * CH, CH)]],
                    rows_v.at[pl.ds(j * CH, CH)],
                    sem,
                )
            )
        for c in copies:
            c.wait()
        pltpu.sync_copy(rows_v, out_hbm.at[pl.ds(wid * PER_W, PER_W)])

    return gather_k(table, idx1d)


BLKH = 1024
CPAIR_T = CPAIR // L
NPAIR_OUT = NPATH // 2


def _gru_body_first(x1_ref, x2_ref, x3_ref, wih_ref, whh_ref, bih_ref, bhh_ref,
                    out_ref):
    _gru_body(x1_ref, x2_ref, x3_ref, wih_ref, whh_ref, bih_ref, bhh_ref,
              None, out_ref)


def _gru_body(x1_ref, x2_ref, x3_ref, wih_ref, whh_ref, bih_ref, bhh_ref,
              prev_ref, out_ref):
    del prev_ref
    wih = wih_ref[...]
    whh = whh_ref[...]
    bih = bih_ref[...]
    bhh = bhh_ref[...]
    G = 2 * H

    def step(x, h):
        gi = jnp.dot(x, wih, preferred_element_type=jnp.float32) + bih
        gh = jnp.dot(h, whh, preferred_element_type=jnp.float32) + bhh
        r = jax.nn.sigmoid(gi[:, 0:G] + gh[:, 0:G])
        z = jax.nn.sigmoid(gi[:, G:2 * G] + gh[:, G:2 * G])
        n = jnp.tanh(gi[:, 2 * G:] + r * gh[:, 2 * G:])
        return (1.0 - z) * n + z * h

    h = jnp.zeros((BLKH, G), dtype=jnp.float32)
    h = step(x1_ref[...], h)
    h = step(x2_ref[...], h)
    h = step(x3_ref[...], h)
    out_ref[...] = jnp.concatenate([h[:, 0:H].T, h[:, H:2 * H].T], axis=1)


def _tc_gru_chunk(x_c, wih2, whh2, bih2, bhh2, prev, chunk):
    nblk = CPAIR_T // BLKH
    base = chunk * nblk
    in_specs = [
        pl.BlockSpec((BLKH, 2 * E), lambda i: (i, 0)),
        pl.BlockSpec((BLKH, 2 * E), lambda i: (i + CPAIR_T // BLKH, 0)),
        pl.BlockSpec((BLKH, 2 * E), lambda i: (i + 2 * (CPAIR_T // BLKH), 0)),
        pl.BlockSpec((2 * H, 6 * H), lambda i: (0, 0)),
        pl.BlockSpec((2 * H, 6 * H), lambda i: (0, 0)),
        pl.BlockSpec((1, 6 * H), lambda i: (0, 0)),
        pl.BlockSpec((1, 6 * H), lambda i: (0, 0)),
    ]
    args = [x_c, x_c, x_c, wih2, whh2, bih2, bhh2]
    aliases = {}
    body = _gru_body_first
    if prev is not None:
        in_specs.append(pl.BlockSpec(memory_space=pl.ANY))
        args.append(prev)
        aliases = {7: 0}
        body = _gru_body
    return pl.pallas_call(
        body,
        grid=(nblk,),
        in_specs=in_specs,
        out_specs=pl.BlockSpec((H, 2 * BLKH), lambda i, b=base: (0, i + b)),
        out_shape=jax.ShapeDtypeStruct((H, NPATH), jnp.float32),
        input_output_aliases=aliases,
        compiler_params=pltpu.CompilerParams(
            dimension_semantics=("arbitrary",),
        ),
    )(*args)


def _pair_weights(Wt):
    z = jnp.zeros((Wt.shape[0], H), dtype=Wt.dtype)
    cols = []
    for g in range(3):
        Wg = Wt[:, g * H:(g + 1) * H]
        cols.append(jnp.concatenate([jnp.concatenate([Wg, z], axis=1),
                                     jnp.concatenate([z, Wg], axis=1)], axis=0))
    return jnp.concatenate(cols, axis=1)


def _pair_bias(b):
    parts = []
    for g in range(3):
        bg = b[g * H:(g + 1) * H]
        parts.append(jnp.concatenate([bg, bg]))
    return jnp.concatenate(parts).reshape(1, 6 * H)


def kernel(support_path, support_pair, support_path_entity, support_relation_set,
           ent_emb_weight, rel_emb_weight, W_ih, W_hh, b_ih, b_hh):
    flat = support_path.reshape(NPATH, L).astype(jnp.int32)
    wih2 = _pair_weights(W_ih.T)
    whh2 = _pair_weights(W_hh.T)
    bih2 = _pair_bias(b_ih)
    bhh2 = _pair_bias(b_hh)

    nblk_c = (CPATH // 2) // BLKH
    idx_all = (flat.reshape(NCHUNK, CPATH, L).transpose(0, 2, 1)
               .reshape(NCHUNK, L, nblk_c, 2, BLKH)
               .transpose(0, 1, 2, 4, 3).reshape(NCHUNK, CIDX))
    out = None
    for c in range(NCHUNK):
        x_c = _sc_gather(rel_emb_weight, idx_all[c]).reshape(CPAIR, 2 * E)
        out = _tc_gru_chunk(x_c, wih2, whh2, bih2, bhh2, out, c)
    return out.T


# speedup vs baseline: 1.5854x; 1.5854x over previous
import functools

import jax
import jax.numpy as jnp
from jax import lax
from jax.experimental import pallas as pl
from jax.experimental.pallas import tpu as pltpu
from jax.experimental.pallas import tpu_sc as plsc

B, P, L = 1024, 16, 3
E, H = 64, 64
NPATH = B * P
NCHUNK = 2
CPATH = NPATH // NCHUNK
CIDX = CPATH * L
CPAIR = CIDX // 2


_NC, _NS = 2, 16
NW = 24
PER_W = CIDX // NW
HALF_W = PER_W // 2
CH = 128
NCH = PER_W // CH
LANES = 16


def _sc_gather(table, idx1d):
    mesh = plsc.VectorSubcoreMesh(core_axis_name="c", subcore_axis_name="s")

    @functools.partial(
        pl.kernel,
        mesh=mesh,
        out_type=jax.ShapeDtypeStruct((CIDX, E), jnp.float32),
        scratch_types=[
            pltpu.VMEM((PER_W,), jnp.int32),
            pltpu.VMEM((PER_W,), jnp.int32),
            pltpu.VMEM((PER_W, E), jnp.float32),
            pltpu.SemaphoreType.DMA,
        ],
        compiler_params=pltpu.CompilerParams(use_tc_tiling_on_sc=False,
                                             needs_layout_passes=False),
    )
    def gather_k(table_hbm, idx_hbm, out_hbm, idx_v, idx2_v, rows_v, sem):
        wid = lax.axis_index("s") * _NC + lax.axis_index("c")

        @pl.when(wid < NW)
        def _():
            pltpu.sync_copy(idx_hbm.at[pl.ds(wid * PER_W, PER_W)], idx_v)
            lane = lax.iota(jnp.int32, LANES)
            for h in range(2):
                for k in range(HALF_W // LANES):
                    vals = idx_v[pl.ds(h * HALF_W + k * LANES, LANES)]
                    plsc.store_scatter(idx2_v, [2 * lane + 2 * k * LANES + h],
                                       vals)
            copies = []
            for j in range(NCH):
                copies.append(
                    pltpu.async_copy(
                        table_hbm.at[idx2_v.at[pl.ds(j * CH, CH)]],
                        rows_v.at[pl.ds(j * CH, CH)],
                        sem,
                    )
                )
            for c in copies:
                c.wait()
            pltpu.sync_copy(rows_v, out_hbm.at[pl.ds(wid * PER_W, PER_W)])

    return gather_k(table, idx1d)


BLKH = 1024
CPAIR_T = CPAIR // L
NPAIR_OUT = NPATH // 2


def _gru_body_first(x1_ref, x2_ref, x3_ref, wih_ref, whh_ref, bih_ref, bhh_ref,
                    out_ref):
    _gru_body(x1_ref, x2_ref, x3_ref, wih_ref, whh_ref, bih_ref, bhh_ref,
              None, out_ref)


def _gru_body(x1_ref, x2_ref, x3_ref, wih_ref, whh_ref, bih_ref, bhh_ref,
              prev_ref, out_ref):
    del prev_ref
    wih = wih_ref[...]
    whh = whh_ref[...]
    bih = bih_ref[...]
    bhh = bhh_ref[...]
    G = 2 * H

    def step(x, h):
        gi = jnp.dot(x, wih, preferred_element_type=jnp.float32) + bih
        gh = jnp.dot(h, whh, preferred_element_type=jnp.float32) + bhh
        r = jax.nn.sigmoid(gi[:, 0:G] + gh[:, 0:G])
        z = jax.nn.sigmoid(gi[:, G:2 * G] + gh[:, G:2 * G])
        n = jnp.tanh(gi[:, 2 * G:] + r * gh[:, 2 * G:])
        return (1.0 - z) * n + z * h

    h = jnp.zeros((BLKH, G), dtype=jnp.float32)
    h = step(x1_ref[...], h)
    h = step(x2_ref[...], h)
    h = step(x3_ref[...], h)
    S = HALF_W
    out_ref[...] = jnp.concatenate(
        [h[0:S, 0:H].T, h[0:S, H:2 * H].T,
         h[S:2 * S, 0:H].T, h[S:2 * S, H:2 * H].T], axis=1)


def _tc_gru_chunk(x_c, wih2, whh2, bih2, bhh2, prev, chunk):
    nblk = CPAIR_T // BLKH
    base = chunk * nblk
    in_specs = [
        pl.BlockSpec((BLKH, 2 * E), lambda i: (i, 0)),
        pl.BlockSpec((BLKH, 2 * E), lambda i: (i + CPAIR_T // BLKH, 0)),
        pl.BlockSpec((BLKH, 2 * E), lambda i: (i + 2 * (CPAIR_T // BLKH), 0)),
        pl.BlockSpec((2 * H, 6 * H), lambda i: (0, 0)),
        pl.BlockSpec((2 * H, 6 * H), lambda i: (0, 0)),
        pl.BlockSpec((1, 6 * H), lambda i: (0, 0)),
        pl.BlockSpec((1, 6 * H), lambda i: (0, 0)),
    ]
    args = [x_c, x_c, x_c, wih2, whh2, bih2, bhh2]
    aliases = {}
    body = _gru_body_first
    if prev is not None:
        in_specs.append(pl.BlockSpec(memory_space=pl.ANY))
        args.append(prev)
        aliases = {7: 0}
        body = _gru_body
    return pl.pallas_call(
        body,
        grid=(nblk,),
        in_specs=in_specs,
        out_specs=pl.BlockSpec((H, 2 * BLKH), lambda i, b=base: (0, i + b)),
        out_shape=jax.ShapeDtypeStruct((H, NPATH), jnp.float32),
        input_output_aliases=aliases,
        compiler_params=pltpu.CompilerParams(
            dimension_semantics=("arbitrary",),
        ),
    )(*args)


def _pair_weights(Wt):
    z = jnp.zeros((Wt.shape[0], H), dtype=Wt.dtype)
    cols = []
    for g in range(3):
        Wg = Wt[:, g * H:(g + 1) * H]
        cols.append(jnp.concatenate([jnp.concatenate([Wg, z], axis=1),
                                     jnp.concatenate([z, Wg], axis=1)], axis=0))
    return jnp.concatenate(cols, axis=1)


def _pair_bias(b):
    parts = []
    for g in range(3):
        bg = b[g * H:(g + 1) * H]
        parts.append(jnp.concatenate([bg, bg]))
    return jnp.concatenate(parts).reshape(1, 6 * H)


def kernel(support_path, support_pair, support_path_entity, support_relation_set,
           ent_emb_weight, rel_emb_weight, W_ih, W_hh, b_ih, b_hh):
    flat = support_path.reshape(NPATH, L).astype(jnp.int32)
    wih2 = _pair_weights(W_ih.T)
    whh2 = _pair_weights(W_hh.T)
    bih2 = _pair_bias(b_ih)
    bhh2 = _pair_bias(b_hh)

    idx_all = (flat.reshape(NCHUNK, CPATH, L).transpose(0, 2, 1)
               .reshape(NCHUNK, CIDX))
    out = None
    for c in range(NCHUNK):
        x_c = _sc_gather(rel_emb_weight, idx_all[c]).reshape(CPAIR, 2 * E)
        out = _tc_gru_chunk(x_c, wih2, whh2, bih2, bhh2, out, c)
    return out.T
